# baseline (speedup 1.0000x reference)
import functools

import jax
import jax.numpy as jnp
from jax import lax
from jax.experimental import pallas as pl
from jax.experimental.pallas import tpu as pltpu
from jax.experimental.pallas import tpu_sc as plsc

H = 128
N_NODES = 10000
N_EDGES = 160000
N_CN = 9000
N_VM = 1000
FEAT_ROWS = 50000

NC = 2
NS = 16
NW = NC * NS

T = 10240
SHIFT = 216
CN_PAD = 9216
E_PAD = 163840
VM_PAD = 1024
SENT = 10000

EC = 64
N_CHUNK = E_PAD // EC
CPW = N_CHUNK // NW
RPT = T // NS

_mesh = plsc.VectorSubcoreMesh(core_axis_name="c", subcore_axis_name="s")
f32 = jnp.float32
i32 = jnp.int32


def _wid():
    return lax.axis_index("s") * NC + lax.axis_index("c")


def _remap_inplace(buf, nvec):
    def body(i, _):
        v = buf[pl.ds(i * 16, 16)]
        buf[pl.ds(i * 16, 16)] = jnp.where(v >= N_CN, v + SHIFT, v)
        return 0
    lax.fori_loop(0, nvec, body, 0)


@functools.partial(
    pl.kernel,
    out_type=(
        jax.ShapeDtypeStruct((T, H), f32),
        jax.ShapeDtypeStruct((E_PAD,), i32),
        jax.ShapeDtypeStruct((E_PAD,), i32),
    ),
    mesh=_mesh,
    scratch_types=(
        pltpu.VMEM((96,), i32),
        pltpu.VMEM((96, H), f32),
        pltpu.VMEM((200, H), f32),
        pltpu.VMEM((1024,), i32),
    ),
)
def _k1(feat, cnp, virt, srcp, dstp, table, srcm, dstm, cidx, crows, vbuf, ibuf):
    wid = _wid()
    for k in range(3):
        base = wid * 288 + k * 96
        pltpu.sync_copy(cnp.at[pl.ds(base, 96)], cidx)
        pltpu.sync_copy(feat.at[cidx], crows)
        pltpu.sync_copy(crows, table.at[pl.ds(base, 96)])
    @pl.when(wid < 5)
    def _():
        vb = wid * 200
        pltpu.sync_copy(virt.at[pl.ds(vb, 200)], vbuf)
        pltpu.sync_copy(vbuf, table.at[pl.ds(CN_PAD + vb, 200)])
    for arr_in, arr_out in ((srcp, srcm), (dstp, dstm)):
        for k in range(5):
            base = wid * 5120 + k * 1024
            pltpu.sync_copy(arr_in.at[pl.ds(base, 1024)], ibuf)
            _remap_inplace(ibuf, 64)
            pltpu.sync_copy(ibuf, arr_out.at[pl.ds(base, 1024)])


NB = 4

FAST_C = 0
BLK = 40
NBLK = N_CHUNK // BLK


def _make_round(with_deg):
    outs = [jax.ShapeDtypeStruct((2 * T, H), f32)]
    scratch = (
        [pltpu.VMEM((BLK, EC), i32),
         pltpu.VMEM((BLK, EC), i32)]
        + [pltpu.VMEM((EC, H), f32) for _ in range(NB)]
        + [pltpu.VMEM((EC,), f32)]
        + [pltpu.VMEM((RPT,), f32)]
        + [pltpu.VMEM_SHARED((T, H), f32),
           pltpu.VMEM_SHARED((T,), f32)]
        + [pltpu.SemaphoreType.DMA for _ in range(3 * NB)]
    )
    if with_deg:
        outs.append(jax.ShapeDtypeStruct((2 * T,), f32))

    def body(table, srcm3, dstm3, *rest):
        if with_deg:
            accp, degp = rest[0], rest[1]
            rest = rest[2:]
        else:
            accp = rest[0]
            rest = rest[1:]
        sidxa, didxa = rest[0], rest[1]
        rows = rest[2:2 + NB]
        ones_v = rest[2 + NB]
        zvec_v = rest[3 + NB]
        acc_sh = rest[4 + NB]
        deg_sh = rest[5 + NB]
        gsem = rest[6 + NB:6 + 2 * NB]
        ssem = rest[6 + 2 * NB:6 + 3 * NB]
        osem = rest[6 + 3 * NB:6 + 4 * NB]
        c = lax.axis_index("c")
        s = lax.axis_index("s")
        wid = s * NC + c
        def zline(i, _):
            rows[0][i // (H // 16), pl.ds((i % (H // 16)) * 16, 16)] = (
                jnp.zeros((16,), f32))
            return 0
        lax.fori_loop(0, EC * H // 16, zline, 0)
        for q in range(RPT // EC):
            pltpu.sync_copy(rows[0], acc_sh.at[pl.ds(s * RPT + q * EC, EC)])
        if with_deg:
            def zv(i, _):
                zvec_v[pl.ds(i * 16, 16)] = jnp.zeros((16,), f32)
                return 0
            lax.fori_loop(0, RPT // 16, zv, 0)
            pltpu.sync_copy(zvec_v, deg_sh.at[pl.ds(s * RPT, RPT)])
            def setones(i, _):
                ones_v[pl.ds(i * 16, 16)] = jnp.full((16,), 1.0, f32)
                return 0
            lax.fori_loop(0, EC // 16, setones, 0)
        plsc.subcore_barrier()

        def drain_all():
            for b in range(NB):
                pltpu.make_async_copy(
                    rows[b], acc_sh.at[didxa.at[b]], ssem[b]).wait()
                if with_deg:
                    pltpu.make_async_copy(
                        ones_v, deg_sh.at[didxa.at[b]], osem[b]).wait()

        def quad(j, _):
            for b in range(NB):
                k = NB * j + b
                @pl.when(j > 0)
                def _(b=b):
                    pltpu.make_async_copy(
                        rows[b], acc_sh.at[didxa.at[b]], ssem[b]).wait()
                    if with_deg:
                        pltpu.make_async_copy(
                            ones_v, deg_sh.at[didxa.at[b]], osem[b]).wait()
                pltpu.async_copy(table.at[sidxa.at[k]], rows[b], gsem[b])
            for b in range(NB):
                k = NB * j + b
                pltpu.make_async_copy(
                    table.at[sidxa.at[k]], rows[b], gsem[b]).wait()
                pltpu.async_copy(rows[b], acc_sh.at[didxa.at[k]], ssem[b],
                                 add=True)
                if with_deg:
                    pltpu.async_copy(ones_v, deg_sh.at[didxa.at[k]],
                                     osem[b], add=True)
            return 0

        def run_block(blk, first):
            if not first:
                drain_all()
            pltpu.sync_copy(srcm3.at[blk], sidxa)
            pltpu.sync_copy(dstm3.at[blk], didxa)
            lax.fori_loop(0, BLK // NB, quad, 0)

        @pl.when(c == FAST_C)
        def _():
            for ph in range(3):
                run_block(3 * s + ph, ph == 0)

        @pl.when(c != FAST_C)
        def _():
            run_block(3 * NS + s, True)

        drain_all()
        plsc.subcore_barrier()
        off = c * T + s * RPT
        pltpu.sync_copy(acc_sh.at[pl.ds(s * RPT, RPT)], accp.at[pl.ds(off, RPT)])
        if with_deg:
            pltpu.sync_copy(deg_sh.at[pl.ds(s * RPT, RPT)], degp.at[pl.ds(off, RPT)])

    return pl.kernel(body, out_type=tuple(outs), mesh=_mesh,
                     scratch_types=tuple(scratch))


_round_deg = _make_round(True)
_round_nodeg = _make_round(False)


_BR = 2048


def _combine1_body(a0, a1, d0, d1, new_ref, inv_ref):
    d = d0[...] + d1[...]
    iv = jnp.where(d > 0, 1.0 / d, 0.0)
    inv_ref[...] = iv
    new_ref[...] = (a0[...] + a1[...]) * iv


def _combine2_body(b0, b1, iv, fin_ref):
    fin_ref[...] = (b0[...] + b1[...]) * iv[...]


def _combine1(accp, degp2):
    row_spec = pl.BlockSpec((_BR, H), lambda i: (i, 0))
    row2_spec = pl.BlockSpec((_BR, H), lambda i: (i + T // _BR, 0))
    deg_spec = pl.BlockSpec((_BR, 1), lambda i: (i, 0))
    deg2_spec = pl.BlockSpec((_BR, 1), lambda i: (i + T // _BR, 0))
    return pl.pallas_call(
        _combine1_body,
        grid=(T // _BR,),
        in_specs=[row_spec, row2_spec, deg_spec, deg2_spec],
        out_specs=[pl.BlockSpec((_BR, H), lambda i: (i, 0)),
                   pl.BlockSpec((_BR, 1), lambda i: (i, 0))],
        out_shape=[jax.ShapeDtypeStruct((T, H), f32),
                   jax.ShapeDtypeStruct((T, 1), f32)],
    )(accp, accp, degp2, degp2)


def _combine2(accp2, inv):
    row_spec = pl.BlockSpec((_BR, H), lambda i: (i, 0))
    row2_spec = pl.BlockSpec((_BR, H), lambda i: (i + T // _BR, 0))
    inv_spec = pl.BlockSpec((_BR, 1), lambda i: (i, 0))
    return pl.pallas_call(
        _combine2_body,
        grid=(T // _BR,),
        in_specs=[row_spec, row2_spec, inv_spec],
        out_specs=pl.BlockSpec((_BR, H), lambda i: (i, 0)),
        out_shape=jax.ShapeDtypeStruct((T, H), f32),
    )(accp2, accp2, inv)


@functools.partial(
    pl.kernel,
    out_type=jax.ShapeDtypeStruct((VM_PAD, H), f32),
    mesh=_mesh,
    scratch_types=(
        pltpu.VMEM((32,), i32),
        pltpu.VMEM((32, H), f32),
    ),
)
def _k6(ftab, vmp, out, vidx, vrows):
    wid = _wid()
    base = wid * 32
    pltpu.sync_copy(vmp.at[pl.ds(base, 32)], vidx)
    _remap_inplace(vidx, 2)
    pltpu.sync_copy(ftab.at[vidx], vrows)
    pltpu.sync_copy(vrows, out.at[pl.ds(base, 32)])


def kernel(features, edge_index, cnodes, vm_idx, virtue_weight):
    src_p = jnp.concatenate(
        [edge_index[0], jnp.zeros((E_PAD - N_EDGES,), i32)])
    dst_p = jnp.concatenate(
        [edge_index[1], jnp.full((E_PAD - N_EDGES,), SENT, i32)])
    cn_p = jnp.concatenate([cnodes, jnp.zeros((CN_PAD - N_CN,), i32)])
    vm_p = jnp.concatenate([vm_idx, jnp.zeros((VM_PAD - N_VM,), i32)])

    table, srcm, dstm = _k1(features, cn_p, virtue_weight, src_p, dst_p)
    srcm = srcm.reshape(NBLK, BLK, EC)
    dstm = dstm.reshape(NBLK, BLK, EC)
    accp, degp = _round_deg(table, srcm, dstm)
    newt, inv = _combine1(accp, degp.reshape(2 * T, 1))
    (accp2,) = _round_nodeg(newt, srcm, dstm)
    fint = _combine2(accp2, inv)
    out = _k6(fint, vm_p)
    return jnp.broadcast_to(out[:N_VM, None, :], (N_VM, 2, H))


# speedup vs baseline: 1.0029x; 1.0029x over previous
import functools

import jax
import jax.numpy as jnp
from jax import lax
from jax.experimental import pallas as pl
from jax.experimental.pallas import tpu as pltpu
from jax.experimental.pallas import tpu_sc as plsc

H = 128
N_NODES = 10000
N_EDGES = 160000
N_CN = 9000
N_VM = 1000
FEAT_ROWS = 50000

NC = 2
NS = 16
NW = NC * NS

T = 10240
SHIFT = 216
CN_PAD = 9216
E_PAD = 163840
VM_PAD = 1024
SENT = 10000

EC = 64
N_CHUNK = E_PAD // EC
CPW = N_CHUNK // NW
RPT = T // NS

_mesh = plsc.VectorSubcoreMesh(core_axis_name="c", subcore_axis_name="s")
f32 = jnp.float32
i32 = jnp.int32


def _wid():
    return lax.axis_index("s") * NC + lax.axis_index("c")


def _remap_inplace(buf, nvec):
    def body(i, _):
        v = buf[pl.ds(i * 16, 16)]
        buf[pl.ds(i * 16, 16)] = jnp.where(v >= N_CN, v + SHIFT, v)
        return 0
    lax.fori_loop(0, nvec, body, 0)


@functools.partial(
    pl.kernel,
    out_type=(
        jax.ShapeDtypeStruct((T, H), f32),
        jax.ShapeDtypeStruct((E_PAD,), i32),
        jax.ShapeDtypeStruct((E_PAD,), i32),
    ),
    mesh=_mesh,
    scratch_types=(
        pltpu.VMEM((96,), i32),
        pltpu.VMEM((96, H), f32),
        pltpu.VMEM((200, H), f32),
        pltpu.VMEM((1024,), i32),
    ),
)
def _k1(feat, cnp, virt, srcp, dstp, table, srcm, dstm, cidx, crows, vbuf, ibuf):
    wid = _wid()
    for k in range(3):
        base = wid * 288 + k * 96
        pltpu.sync_copy(cnp.at[pl.ds(base, 96)], cidx)
        pltpu.sync_copy(feat.at[cidx], crows)
        pltpu.sync_copy(crows, table.at[pl.ds(base, 96)])
    @pl.when(wid < 5)
    def _():
        vb = wid * 200
        pltpu.sync_copy(virt.at[pl.ds(vb, 200)], vbuf)
        pltpu.sync_copy(vbuf, table.at[pl.ds(CN_PAD + vb, 200)])
    for arr_in, arr_out in ((srcp, srcm), (dstp, dstm)):
        for k in range(5):
            base = wid * 5120 + k * 1024
            pltpu.sync_copy(arr_in.at[pl.ds(base, 1024)], ibuf)
            _remap_inplace(ibuf, 64)
            pltpu.sync_copy(ibuf, arr_out.at[pl.ds(base, 1024)])


NB = 4

FAST_C = 0
BLK = 40
NBLK = N_CHUNK // BLK


def _make_round(with_deg):
    outs = [jax.ShapeDtypeStruct((2 * T, H), f32)]
    scratch = (
        [pltpu.VMEM((BLK, EC), i32),
         pltpu.VMEM((BLK, EC), i32)]
        + [pltpu.VMEM((EC, H), f32) for _ in range(NB)]
        + [pltpu.VMEM((EC,), f32)]
        + [pltpu.VMEM((RPT,), f32)]
        + [pltpu.VMEM_SHARED((T, H), f32),
           pltpu.VMEM_SHARED((T,), f32)]
        + [pltpu.SemaphoreType.DMA for _ in range(3 * NB)]
    )
    if with_deg:
        outs.append(jax.ShapeDtypeStruct((2 * T,), f32))

    def body(table, srcm3, dstm3, *rest):
        if with_deg:
            accp, degp = rest[0], rest[1]
            rest = rest[2:]
        else:
            accp = rest[0]
            rest = rest[1:]
        sidxa, didxa = rest[0], rest[1]
        rows = rest[2:2 + NB]
        ones_v = rest[2 + NB]
        zvec_v = rest[3 + NB]
        acc_sh = rest[4 + NB]
        deg_sh = rest[5 + NB]
        gsem = rest[6 + NB:6 + 2 * NB]
        ssem = rest[6 + 2 * NB:6 + 3 * NB]
        osem = rest[6 + 3 * NB:6 + 4 * NB]
        c = lax.axis_index("c")
        s = lax.axis_index("s")
        wid = s * NC + c
        def zline(i, _):
            rows[0][i // (H // 16), pl.ds((i % (H // 16)) * 16, 16)] = (
                jnp.zeros((16,), f32))
            return 0
        lax.fori_loop(0, EC * H // 16, zline, 0)
        for q in range(RPT // EC):
            pltpu.sync_copy(rows[0], acc_sh.at[pl.ds(s * RPT + q * EC, EC)])
        if with_deg:
            def zv(i, _):
                zvec_v[pl.ds(i * 16, 16)] = jnp.zeros((16,), f32)
                return 0
            lax.fori_loop(0, RPT // 16, zv, 0)
            pltpu.sync_copy(zvec_v, deg_sh.at[pl.ds(s * RPT, RPT)])
            def setones(i, _):
                ones_v[pl.ds(i * 16, 16)] = jnp.full((16,), 1.0, f32)
                return 0
            lax.fori_loop(0, EC // 16, setones, 0)
        plsc.subcore_barrier()

        def drain_all():
            for b in range(NB):
                pltpu.make_async_copy(
                    rows[b], acc_sh.at[didxa.at[b]], ssem[b]).wait()
                if with_deg:
                    pltpu.make_async_copy(
                        ones_v, deg_sh.at[didxa.at[b]], osem[b]).wait()

        def quad(j, _):
            for b in range(NB):
                k = NB * j + b
                @pl.when(j > 0)
                def _(b=b):
                    pltpu.make_async_copy(
                        rows[b], acc_sh.at[didxa.at[b]], ssem[b]).wait()
                    if with_deg:
                        pltpu.make_async_copy(
                            ones_v, deg_sh.at[didxa.at[b]], osem[b]).wait()
                pltpu.async_copy(table.at[sidxa.at[k]], rows[b], gsem[b])
            for b in range(NB):
                k = NB * j + b
                pltpu.make_async_copy(
                    table.at[sidxa.at[k]], rows[b], gsem[b]).wait()
                pltpu.async_copy(rows[b], acc_sh.at[didxa.at[k]], ssem[b],
                                 add=True)
                if with_deg:
                    pltpu.async_copy(ones_v, deg_sh.at[didxa.at[k]],
                                     osem[b], add=True)
            return 0

        def run_block(blk, first):
            if not first:
                drain_all()
            pltpu.sync_copy(srcm3.at[blk], sidxa)
            pltpu.sync_copy(dstm3.at[blk], didxa)
            lax.fori_loop(0, BLK // NB, quad, 0)

        @pl.when(c == FAST_C)
        def _():
            for ph in range(3):
                run_block(3 * s + ph, ph == 0)

        @pl.when(c != FAST_C)
        def _():
            run_block(3 * NS + s, True)

        drain_all()
        plsc.subcore_barrier()
        off = c * T + s * RPT
        for q in range(RPT // EC):
            b = q % NB
            if q >= NB:
                pltpu.make_async_copy(
                    rows[b], accp.at[pl.ds(off, EC)], gsem[b]).wait()
            pltpu.sync_copy(acc_sh.at[pl.ds(s * RPT + q * EC, EC)], rows[b])
            pltpu.async_copy(rows[b], accp.at[pl.ds(off + q * EC, EC)], gsem[b])
        for b in range(NB):
            pltpu.make_async_copy(
                rows[b], accp.at[pl.ds(off, EC)], gsem[b]).wait()
        if with_deg:
            pltpu.sync_copy(deg_sh.at[pl.ds(s * RPT, RPT)], degp.at[pl.ds(off, RPT)])

    return pl.kernel(body, out_type=tuple(outs), mesh=_mesh,
                     scratch_types=tuple(scratch))


_round_deg = _make_round(True)
_round_nodeg = _make_round(False)


_BR = 2048


def _combine1_body(a0, a1, d0, d1, new_ref, inv_ref):
    d = d0[...] + d1[...]
    iv = jnp.where(d > 0, 1.0 / d, 0.0)
    inv_ref[...] = iv
    new_ref[...] = (a0[...] + a1[...]) * iv


def _combine2_body(b0, b1, iv, fin_ref):
    fin_ref[...] = (b0[...] + b1[...]) * iv[...]


def _combine1(accp, degp2):
    row_spec = pl.BlockSpec((_BR, H), lambda i: (i, 0))
    row2_spec = pl.BlockSpec((_BR, H), lambda i: (i + T // _BR, 0))
    deg_spec = pl.BlockSpec((_BR, 1), lambda i: (i, 0))
    deg2_spec = pl.BlockSpec((_BR, 1), lambda i: (i + T // _BR, 0))
    return pl.pallas_call(
        _combine1_body,
        grid=(T // _BR,),
        in_specs=[row_spec, row2_spec, deg_spec, deg2_spec],
        out_specs=[pl.BlockSpec((_BR, H), lambda i: (i, 0)),
                   pl.BlockSpec((_BR, 1), lambda i: (i, 0))],
        out_shape=[jax.ShapeDtypeStruct((T, H), f32),
                   jax.ShapeDtypeStruct((T, 1), f32)],
    )(accp, accp, degp2, degp2)


def _combine2(accp2, inv):
    row_spec = pl.BlockSpec((_BR, H), lambda i: (i, 0))
    row2_spec = pl.BlockSpec((_BR, H), lambda i: (i + T // _BR, 0))
    inv_spec = pl.BlockSpec((_BR, 1), lambda i: (i, 0))
    return pl.pallas_call(
        _combine2_body,
        grid=(T // _BR,),
        in_specs=[row_spec, row2_spec, inv_spec],
        out_specs=pl.BlockSpec((_BR, H), lambda i: (i, 0)),
        out_shape=jax.ShapeDtypeStruct((T, H), f32),
    )(accp2, accp2, inv)


@functools.partial(
    pl.kernel,
    out_type=jax.ShapeDtypeStruct((VM_PAD, H), f32),
    mesh=_mesh,
    scratch_types=(
        pltpu.VMEM((32,), i32),
        pltpu.VMEM((32, H), f32),
    ),
)
def _k6(ftab, vmp, out, vidx, vrows):
    wid = _wid()
    base = wid * 32
    pltpu.sync_copy(vmp.at[pl.ds(base, 32)], vidx)
    _remap_inplace(vidx, 2)
    pltpu.sync_copy(ftab.at[vidx], vrows)
    pltpu.sync_copy(vrows, out.at[pl.ds(base, 32)])


def kernel(features, edge_index, cnodes, vm_idx, virtue_weight):
    src_p = jnp.concatenate(
        [edge_index[0], jnp.zeros((E_PAD - N_EDGES,), i32)])
    dst_p = jnp.concatenate(
        [edge_index[1], jnp.full((E_PAD - N_EDGES,), SENT, i32)])
    cn_p = jnp.concatenate([cnodes, jnp.zeros((CN_PAD - N_CN,), i32)])
    vm_p = jnp.concatenate([vm_idx, jnp.zeros((VM_PAD - N_VM,), i32)])

    table, srcm, dstm = _k1(features, cn_p, virtue_weight, src_p, dst_p)
    srcm = srcm.reshape(NBLK, BLK, EC)
    dstm = dstm.reshape(NBLK, BLK, EC)
    accp, degp = _round_deg(table, srcm, dstm)
    newt, inv = _combine1(accp, degp.reshape(2 * T, 1))
    (accp2,) = _round_nodeg(newt, srcm, dstm)
    fint = _combine2(accp2, inv)
    out = _k6(fint, vm_p)
    return jnp.broadcast_to(out[:N_VM, None, :], (N_VM, 2, H))


# speedup vs baseline: 1.1420x; 1.1387x over previous
import functools

import jax
import jax.numpy as jnp
from jax import lax
from jax.experimental import pallas as pl
from jax.experimental.pallas import tpu as pltpu
from jax.experimental.pallas import tpu_sc as plsc

H = 128
N_NODES = 10000
N_EDGES = 160000
N_CN = 9000
N_VM = 1000
FEAT_ROWS = 50000

NC = 2
NS = 16
NW = NC * NS

T = 10240
SHIFT = 216
CN_PAD = 9216
E_PAD = 163840
VM_PAD = 1024
SENT = 10000

EC = 64
N_CHUNK = E_PAD // EC
CPW = N_CHUNK // NW
RPT = T // NS

_mesh = plsc.VectorSubcoreMesh(core_axis_name="c", subcore_axis_name="s")
f32 = jnp.float32
i32 = jnp.int32


def _wid():
    return lax.axis_index("s") * NC + lax.axis_index("c")


def _remap_inplace(buf, nvec):
    def body(i, _):
        v = buf[pl.ds(i * 16, 16)]
        buf[pl.ds(i * 16, 16)] = jnp.where(v >= N_CN, v + SHIFT, v)
        return 0
    lax.fori_loop(0, nvec, body, 0)


@functools.partial(
    pl.kernel,
    out_type=(
        jax.ShapeDtypeStruct((T, H), f32),
        jax.ShapeDtypeStruct((E_PAD,), i32),
        jax.ShapeDtypeStruct((E_PAD,), i32),
    ),
    mesh=_mesh,
    scratch_types=(
        pltpu.VMEM((96,), i32),
        pltpu.VMEM((96, H), f32),
        pltpu.VMEM((200, H), f32),
        pltpu.VMEM((1024,), i32),
    ),
)
def _k1(feat, cnp, virt, srcp, dstp, table, srcm, dstm, cidx, crows, vbuf, ibuf):
    wid = _wid()
    for k in range(3):
        base = wid * 288 + k * 96
        pltpu.sync_copy(cnp.at[pl.ds(base, 96)], cidx)
        pltpu.sync_copy(feat.at[cidx], crows)
        pltpu.sync_copy(crows, table.at[pl.ds(base, 96)])
    @pl.when(wid < 5)
    def _():
        vb = wid * 200
        pltpu.sync_copy(virt.at[pl.ds(vb, 200)], vbuf)
        pltpu.sync_copy(vbuf, table.at[pl.ds(CN_PAD + vb, 200)])
    for arr_in, arr_out in ((srcp, srcm), (dstp, dstm)):
        for k in range(5):
            base = wid * 5120 + k * 1024
            pltpu.sync_copy(arr_in.at[pl.ds(base, 1024)], ibuf)
            _remap_inplace(ibuf, 64)
            pltpu.sync_copy(ibuf, arr_out.at[pl.ds(base, 1024)])


NB = 4

FAST_C = 0
BLKC = 16
FBLKS = 9
SBLKS = 1
SLOW_BASE = NS * FBLKS * BLKC


def _make_round(with_deg):
    outs = [jax.ShapeDtypeStruct((2 * T, H), f32)]
    scratch = (
        [pltpu.VMEM((BLKC, EC), i32),
         pltpu.VMEM((BLKC, EC), i32)]
        + [pltpu.VMEM((EC, H), f32) for _ in range(NB)]
        + [pltpu.VMEM((EC,), f32)]
        + [pltpu.VMEM((RPT,), f32)]
        + [pltpu.VMEM_SHARED((T, H), f32),
           pltpu.VMEM_SHARED((T,), f32)]
        + [pltpu.SemaphoreType.DMA for _ in range(3 * NB)]
    )
    if with_deg:
        outs.append(jax.ShapeDtypeStruct((2 * T,), f32))

    def body(table, srcm3, dstm3, *rest):
        if with_deg:
            accp, degp = rest[0], rest[1]
            rest = rest[2:]
        else:
            accp = rest[0]
            rest = rest[1:]
        sidxa, didxa = rest[0], rest[1]
        rows = rest[2:2 + NB]
        ones_v = rest[2 + NB]
        zvec_v = rest[3 + NB]
        acc_sh = rest[4 + NB]
        deg_sh = rest[5 + NB]
        gsem = rest[6 + NB:6 + 2 * NB]
        ssem = rest[6 + 2 * NB:6 + 3 * NB]
        osem = rest[6 + 3 * NB:6 + 4 * NB]
        c = lax.axis_index("c")
        s = lax.axis_index("s")
        wid = s * NC + c
        def zline(i, _):
            rows[0][i // (H // 16), pl.ds((i % (H // 16)) * 16, 16)] = (
                jnp.zeros((16,), f32))
            return 0
        lax.fori_loop(0, EC * H // 16, zline, 0)
        for q in range(RPT // EC):
            pltpu.sync_copy(rows[0], acc_sh.at[pl.ds(s * RPT + q * EC, EC)])
        if with_deg:
            def zv(i, _):
                zvec_v[pl.ds(i * 16, 16)] = jnp.zeros((16,), f32)
                return 0
            lax.fori_loop(0, RPT // 16, zv, 0)
            pltpu.sync_copy(zvec_v, deg_sh.at[pl.ds(s * RPT, RPT)])
            def setones(i, _):
                ones_v[pl.ds(i * 16, 16)] = jnp.full((16,), 1.0, f32)
                return 0
            lax.fori_loop(0, EC // 16, setones, 0)
        plsc.subcore_barrier()

        def drain_all():
            for b in range(NB):
                pltpu.make_async_copy(
                    rows[b], acc_sh.at[didxa.at[b]], ssem[b]).wait()
                if with_deg:
                    pltpu.make_async_copy(
                        ones_v, deg_sh.at[didxa.at[b]], osem[b]).wait()

        def quad(j, _):
            for b in range(NB):
                k = NB * j + b
                @pl.when(j > 0)
                def _(b=b):
                    pltpu.make_async_copy(
                        rows[b], acc_sh.at[didxa.at[b]], ssem[b]).wait()
                    if with_deg:
                        pltpu.make_async_copy(
                            ones_v, deg_sh.at[didxa.at[b]], osem[b]).wait()
                pltpu.async_copy(table.at[sidxa.at[k]], rows[b], gsem[b])
            for b in range(NB):
                k = NB * j + b
                pltpu.make_async_copy(
                    table.at[sidxa.at[k]], rows[b], gsem[b]).wait()
                pltpu.async_copy(rows[b], acc_sh.at[didxa.at[k]], ssem[b],
                                 add=True)
                if with_deg:
                    pltpu.async_copy(ones_v, deg_sh.at[didxa.at[k]],
                                     osem[b], add=True)
            return 0

        def run_block(base, first):
            if not first:
                drain_all()
            pltpu.sync_copy(srcm3.at[pl.ds(base, BLKC)], sidxa)
            pltpu.sync_copy(dstm3.at[pl.ds(base, BLKC)], didxa)
            lax.fori_loop(0, BLKC // NB, quad, 0)

        @pl.when(c == FAST_C)
        def _():
            for blk in range(FBLKS):
                run_block(s * (FBLKS * BLKC) + blk * BLKC, blk == 0)

        @pl.when(c != FAST_C)
        def _():
            for blk in range(SBLKS):
                run_block(SLOW_BASE + s * (SBLKS * BLKC) + blk * BLKC,
                          blk == 0)

        drain_all()
        plsc.subcore_barrier()
        off = c * T + s * RPT
        for q in range(RPT // EC):
            b = q % NB
            if q >= NB:
                pltpu.make_async_copy(
                    rows[b], accp.at[pl.ds(off, EC)], gsem[b]).wait()
            pltpu.sync_copy(acc_sh.at[pl.ds(s * RPT + q * EC, EC)], rows[b])
            pltpu.async_copy(rows[b], accp.at[pl.ds(off + q * EC, EC)], gsem[b])
        for b in range(NB):
            pltpu.make_async_copy(
                rows[b], accp.at[pl.ds(off, EC)], gsem[b]).wait()
        if with_deg:
            pltpu.sync_copy(deg_sh.at[pl.ds(s * RPT, RPT)], degp.at[pl.ds(off, RPT)])

    return pl.kernel(body, out_type=tuple(outs), mesh=_mesh,
                     scratch_types=tuple(scratch))


_round_deg = _make_round(True)
_round_nodeg = _make_round(False)


_BR = 2048


def _combine1_body(a0, a1, d0, d1, new_ref, inv_ref):
    d = d0[...] + d1[...]
    iv = jnp.where(d > 0, 1.0 / d, 0.0)
    inv_ref[...] = iv
    new_ref[...] = (a0[...] + a1[...]) * iv


def _combine2_body(b0, b1, iv, fin_ref):
    fin_ref[...] = (b0[...] + b1[...]) * iv[...]


def _combine1(accp, degp2):
    row_spec = pl.BlockSpec((_BR, H), lambda i: (i, 0))
    row2_spec = pl.BlockSpec((_BR, H), lambda i: (i + T // _BR, 0))
    deg_spec = pl.BlockSpec((_BR, 1), lambda i: (i, 0))
    deg2_spec = pl.BlockSpec((_BR, 1), lambda i: (i + T // _BR, 0))
    return pl.pallas_call(
        _combine1_body,
        grid=(T // _BR,),
        in_specs=[row_spec, row2_spec, deg_spec, deg2_spec],
        out_specs=[pl.BlockSpec((_BR, H), lambda i: (i, 0)),
                   pl.BlockSpec((_BR, 1), lambda i: (i, 0))],
        out_shape=[jax.ShapeDtypeStruct((T, H), f32),
                   jax.ShapeDtypeStruct((T, 1), f32)],
    )(accp, accp, degp2, degp2)


def _combine2(accp2, inv):
    row_spec = pl.BlockSpec((_BR, H), lambda i: (i, 0))
    row2_spec = pl.BlockSpec((_BR, H), lambda i: (i + T // _BR, 0))
    inv_spec = pl.BlockSpec((_BR, 1), lambda i: (i, 0))
    return pl.pallas_call(
        _combine2_body,
        grid=(T // _BR,),
        in_specs=[row_spec, row2_spec, inv_spec],
        out_specs=pl.BlockSpec((_BR, H), lambda i: (i, 0)),
        out_shape=jax.ShapeDtypeStruct((T, H), f32),
    )(accp2, accp2, inv)


@functools.partial(
    pl.kernel,
    out_type=jax.ShapeDtypeStruct((VM_PAD, H), f32),
    mesh=_mesh,
    scratch_types=(
        pltpu.VMEM((32,), i32),
        pltpu.VMEM((32, H), f32),
    ),
)
def _k6(ftab, vmp, out, vidx, vrows):
    wid = _wid()
    base = wid * 32
    pltpu.sync_copy(vmp.at[pl.ds(base, 32)], vidx)
    _remap_inplace(vidx, 2)
    pltpu.sync_copy(ftab.at[vidx], vrows)
    pltpu.sync_copy(vrows, out.at[pl.ds(base, 32)])


def kernel(features, edge_index, cnodes, vm_idx, virtue_weight):
    src_p = jnp.concatenate(
        [edge_index[0], jnp.zeros((E_PAD - N_EDGES,), i32)])
    dst_p = jnp.concatenate(
        [edge_index[1], jnp.full((E_PAD - N_EDGES,), SENT, i32)])
    cn_p = jnp.concatenate([cnodes, jnp.zeros((CN_PAD - N_CN,), i32)])
    vm_p = jnp.concatenate([vm_idx, jnp.zeros((VM_PAD - N_VM,), i32)])

    table, srcm, dstm = _k1(features, cn_p, virtue_weight, src_p, dst_p)
    srcm = srcm.reshape(N_CHUNK, EC)
    dstm = dstm.reshape(N_CHUNK, EC)
    accp, degp = _round_deg(table, srcm, dstm)
    newt, inv = _combine1(accp, degp.reshape(2 * T, 1))
    (accp2,) = _round_nodeg(newt, srcm, dstm)
    fint = _combine2(accp2, inv)
    out = _k6(fint, vm_p)
    return jnp.broadcast_to(out[:N_VM, None, :], (N_VM, 2, H))
